# baseline (speedup 1.0000x reference)
import functools

import jax
import jax.numpy as jnp
from jax import lax
from jax.experimental import pallas as pl
from jax.experimental.pallas import tpu as pltpu
from jax.experimental.pallas import tpu_sc as plsc

D = 64
NC = 2
NS = 16
L = 16
NW = NC * NS
B = 16384
B_PER_W = B // NW
CHUNK = 128
NCHUNK = B_PER_W // CHUNK
GROUPS = B_PER_W // L
GPC = CHUNK // L


def _log_sigmoid(x):
    z = jnp.exp(-jnp.abs(x))
    t = z / (z + 2.0)
    t2 = t * t
    p = 1.0 / 9.0 + t2 * (1.0 / 11.0)
    p = 1.0 / 7.0 + t2 * p
    p = 1.0 / 5.0 + t2 * p
    p = 1.0 / 3.0 + t2 * p
    p = 1.0 + t2 * p
    return jnp.minimum(x, 0.0) - 2.0 * t * p


@functools.cache
def _skipgram_sc():
    @functools.partial(
        pl.kernel,
        mesh=plsc.VectorSubcoreMesh(core_axis_name="c", subcore_axis_name="s",
                                    num_cores=NC, num_subcores=NS),
        out_type=jax.ShapeDtypeStruct((NW, L), jnp.float32),
        compiler_params=pltpu.CompilerParams(needs_layout_passes=False,
                                             use_tc_tiling_on_sc=False),
        scratch_types=[
            pltpu.VMEM((NCHUNK, CHUNK), jnp.int32),
            pltpu.VMEM((NCHUNK, CHUNK), jnp.int32),
            pltpu.VMEM((NCHUNK, CHUNK, D), jnp.float32),
            pltpu.VMEM((NCHUNK, CHUNK, D), jnp.float32),
            pltpu.VMEM((L,), jnp.float32),
            pltpu.SemaphoreType.DMA,
        ],
    )
    def body(u_pos2, v_pos2, u_emb, v_emb, out, u_idx, v_idx,
             u_rows, v_rows, out_v, sem):
        wid = lax.axis_index("s") * NC + lax.axis_index("c")
        ib = wid * NCHUNK
        pltpu.sync_copy(u_pos2.at[pl.ds(ib, NCHUNK)], u_idx)
        pltpu.sync_copy(v_pos2.at[pl.ds(ib, NCHUNK)], v_idx)
        copies = []
        for j in range(NCHUNK):
            copies.append(pltpu.async_copy(u_emb.at[u_idx.at[j]], u_rows.at[j], sem))
            copies.append(pltpu.async_copy(v_emb.at[v_idx.at[j]], v_rows.at[j], sem))
        for c in copies:
            c.wait()

        lane = lax.iota(jnp.int32, L)

        def group_body(g, acc):
            jv = jnp.full((L,), g // GPC, jnp.int32)
            rv = lane + (g % GPC) * L
            score = jnp.zeros((L,), jnp.float32)
            for k in range(D):
                kv = jnp.full((L,), k, jnp.int32)
                uk = plsc.load_gather(u_rows, [jv, rv, kv])
                vk = plsc.load_gather(v_rows, [jv, rv, kv])
                score = score + uk * vk
            return acc + _log_sigmoid(score)

        acc = lax.fori_loop(0, GROUPS, group_body, jnp.zeros((L,), jnp.float32))
        out_v[...] = acc
        pltpu.sync_copy(out_v, out.at[wid])

    return body


def kernel(u_pos, v_pos, batch_size, u_embeddings, v_embeddings):
    u2 = u_pos.reshape(B // CHUNK, CHUNK)
    v2 = v_pos.reshape(B // CHUNK, CHUNK)
    partials = _skipgram_sc()(u2, v2, u_embeddings, v_embeddings)
    return -jnp.sum(partials) / batch_size


# speedup vs baseline: 1.1876x; 1.1876x over previous
import functools

import jax
import jax.numpy as jnp
from jax import lax
from jax.experimental import pallas as pl
from jax.experimental.pallas import tpu as pltpu
from jax.experimental.pallas import tpu_sc as plsc

D = 64
NC = 2
NS = 16
L = 16
NW = NC * NS
B = 16384
B_PER_W = B // NW
CHUNK = 128
NCHUNK = B_PER_W // CHUNK
GPC = CHUNK // L


def _log_sigmoid(x):
    z = jnp.exp(-jnp.abs(x))
    t = z / (z + 2.0)
    t2 = t * t
    p = 1.0 / 9.0 + t2 * (1.0 / 11.0)
    p = 1.0 / 7.0 + t2 * p
    p = 1.0 / 5.0 + t2 * p
    p = 1.0 / 3.0 + t2 * p
    p = 1.0 + t2 * p
    return jnp.minimum(x, 0.0) - 2.0 * t * p


@functools.cache
def _skipgram_sc():
    @functools.partial(
        pl.kernel,
        mesh=plsc.VectorSubcoreMesh(core_axis_name="c", subcore_axis_name="s",
                                    num_cores=NC, num_subcores=NS),
        out_type=jax.ShapeDtypeStruct((NW, L), jnp.float32),
        compiler_params=pltpu.CompilerParams(needs_layout_passes=False,
                                             use_tc_tiling_on_sc=False,
                                             disable_bounds_checks=True),
        scratch_types=[
            pltpu.VMEM((NCHUNK, CHUNK), jnp.int32),
            pltpu.VMEM((NCHUNK, CHUNK), jnp.int32),
            pltpu.VMEM((NCHUNK, CHUNK, D), jnp.float32),
            pltpu.VMEM((NCHUNK, CHUNK, D), jnp.float32),
            pltpu.VMEM((L,), jnp.float32),
            pltpu.SemaphoreType.DMA,
            pltpu.SemaphoreType.DMA,
            pltpu.SemaphoreType.DMA,
            pltpu.SemaphoreType.DMA,
        ],
    )
    def body(u_pos2, v_pos2, u_emb, v_emb, out, u_idx, v_idx,
             u_rows, v_rows, out_v, sem0, sem1, sem2, sem3):
        wid = lax.axis_index("s") * NC + lax.axis_index("c")
        ib = wid * NCHUNK
        pltpu.sync_copy(u_pos2.at[pl.ds(ib, NCHUNK)], u_idx)
        pltpu.sync_copy(v_pos2.at[pl.ds(ib, NCHUNK)], v_idx)
        sems = [sem0, sem1, sem2, sem3]
        copies = []
        for j in range(NCHUNK):
            copies.append(
                (pltpu.async_copy(u_emb.at[u_idx.at[j]], u_rows.at[j], sems[j]),
                 pltpu.async_copy(v_emb.at[v_idx.at[j]], v_rows.at[j], sems[j])))

        lane = lax.iota(jnp.int32, L)
        acc = jnp.zeros((L,), jnp.float32)
        for j in range(NCHUNK):
            cu, cv = copies[j]
            cu.wait()
            cv.wait()

            def group_body(g, acc, j=j):
                score = jnp.zeros((L,), jnp.float32)
                for r in range(L):
                    row = g * L + r
                    prod = (u_rows[j, row, pl.ds(0, L)]
                            * v_rows[j, row, pl.ds(0, L)])
                    for c in range(1, D // L):
                        prod = prod + (u_rows[j, row, pl.ds(c * L, L)]
                                       * v_rows[j, row, pl.ds(c * L, L)])
                    total = jnp.sum(prod)
                    score = jnp.where(lane == r, total, score)
                return acc + _log_sigmoid(score)

            acc = lax.fori_loop(0, GPC, group_body, acc)

        out_v[...] = acc
        pltpu.sync_copy(out_v, out.at[wid])

    return body


def kernel(u_pos, v_pos, batch_size, u_embeddings, v_embeddings):
    u2 = u_pos.reshape(B // CHUNK, CHUNK)
    v2 = v_pos.reshape(B // CHUNK, CHUNK)
    partials = _skipgram_sc()(u2, v2, u_embeddings, v_embeddings)
    return -jnp.sum(partials) / batch_size
